# Initial kernel scaffold; baseline (speedup 1.0000x reference)
import jax
import jax.numpy as jnp
from jax.experimental import pallas as pl


def kernel(input_feature, edge_weight, weight, bias, edge_index):
    raise NotImplementedError("write your pallas kernel here")


# speedup vs baseline: 3.4283x; 3.4283x over previous
import functools

import jax
import jax.numpy as jnp
from jax import lax
from jax.experimental import pallas as pl
from jax.experimental.pallas import tpu as pltpu
from jax.experimental.pallas import tpu_sc as plsc

N_NODES = 10000
D_IN = 128
D_OUT = 128
N_EDGES = 320000

NC = 2
NS = 16
LANES = 16

CHUNK = 128
K_PER_TILE = 79
E_PAD = NC * NS * K_PER_TILE * CHUNK
ACC_ROWS = 10240
ROWS_PER_TILE = ACC_ROWS // NS


def _matmul_body(x_ref, w_ref, o_ref):
    o_ref[...] = jnp.dot(x_ref[...], w_ref[...],
                         preferred_element_type=jnp.float32)


def _combine_body(p_ref, b_ref, o_ref):
    o_ref[...] = p_ref[0] + p_ref[1] + b_ref[...]


def _sc_body(support, srcs, dsts, ws, out,
             acc, src_v, dst_v, w_v, rows_v, sem):
    cid = lax.axis_index("c")
    sid = lax.axis_index("s")
    wid = cid * NS + sid

    zvec = jnp.zeros((LANES,), jnp.float32)

    def zero_row(r, _):
        for j in range(D_OUT // LANES):
            rows_v[r, pl.ds(j * LANES, LANES)] = zvec
        return 0

    lax.fori_loop(0, CHUNK, zero_row, 0)
    for t in range(ROWS_PER_TILE // CHUNK):
        pltpu.sync_copy(rows_v,
                        acc.at[pl.ds(sid * ROWS_PER_TILE + t * CHUNK, CHUNK)])
    plsc.subcore_barrier()

    def chunk_body(k, _):
        off = (wid * K_PER_TILE + k) * CHUNK
        pltpu.sync_copy(srcs.at[pl.ds(off, CHUNK)], src_v)
        pltpu.sync_copy(dsts.at[pl.ds(off, CHUNK)], dst_v)
        pltpu.sync_copy(ws.at[pl.ds(off, CHUNK)], w_v)
        pltpu.async_copy(support.at[src_v], rows_v, sem).wait()

        def group_body(g, _):
            wv = w_v[pl.ds(g * LANES, LANES)]
            for i in range(LANES):
                w = wv[i]
                e = g * LANES + i
                for j in range(D_OUT // LANES):
                    sl = pl.ds(j * LANES, LANES)
                    rows_v[e, sl] = rows_v[e, sl] * w
            return 0

        lax.fori_loop(0, CHUNK // LANES, group_body, 0)
        pltpu.sync_copy(rows_v, acc.at[dst_v], add=True)
        return 0

    lax.fori_loop(0, K_PER_TILE, chunk_body, 0)
    plsc.subcore_barrier()

    for t in range(ROWS_PER_TILE // CHUNK):
        r0 = sid * ROWS_PER_TILE + t * CHUNK
        pltpu.sync_copy(acc.at[pl.ds(r0, CHUNK)],
                        out.at[cid].at[pl.ds(r0, CHUNK)])


_sc_edges = functools.partial(
    pl.kernel,
    out_type=jax.ShapeDtypeStruct((NC, ACC_ROWS, D_OUT), jnp.float32),
    mesh=plsc.VectorSubcoreMesh(core_axis_name="c", subcore_axis_name="s"),
    scratch_types=[
        pltpu.VMEM_SHARED((ACC_ROWS, D_OUT), jnp.float32),
        pltpu.VMEM((CHUNK,), jnp.int32),
        pltpu.VMEM((CHUNK,), jnp.int32),
        pltpu.VMEM((CHUNK,), jnp.float32),
        pltpu.VMEM((CHUNK, D_OUT), jnp.float32),
        pltpu.SemaphoreType.DMA,
    ],
)(_sc_body)


@jax.jit
def kernel(input_feature, edge_weight, weight, bias, edge_index):
    support = pl.pallas_call(
        _matmul_body,
        grid=(10,),
        in_specs=[
            pl.BlockSpec((N_NODES // 10, D_IN), lambda i: (i, 0)),
            pl.BlockSpec((D_IN, D_OUT), lambda i: (0, 0)),
        ],
        out_specs=pl.BlockSpec((N_NODES // 10, D_OUT), lambda i: (i, 0)),
        out_shape=jax.ShapeDtypeStruct((N_NODES, D_OUT), jnp.float32),
    )(input_feature, weight)

    pad = E_PAD - N_EDGES
    src = jnp.pad(edge_index[0].astype(jnp.int32), (0, pad))
    dst = jnp.pad(edge_index[1].astype(jnp.int32), (0, pad))
    ew = jnp.pad(edge_weight, (0, pad))

    partials = _sc_edges(support, src, dst, ew)[:, :N_NODES, :]

    out = pl.pallas_call(
        _combine_body,
        grid=(10,),
        in_specs=[
            pl.BlockSpec((NC, N_NODES // 10, D_OUT), lambda i: (0, i, 0)),
            pl.BlockSpec((1, D_OUT), lambda i: (0, 0)),
        ],
        out_specs=pl.BlockSpec((N_NODES // 10, D_OUT), lambda i: (i, 0)),
        out_shape=jax.ShapeDtypeStruct((N_NODES, D_OUT), jnp.float32),
    )(partials, bias.reshape(1, D_OUT))
    return out
